# baseline (speedup 1.0000x reference)
import jax
import jax.numpy as jnp
from jax.experimental import pallas as pl

_NUM_CLASSES = 805
_BLOCK_ROWS = 64


def _onehot_block(x_ref, o_ref):
    o_ref[...] = jnp.zeros(o_ref.shape, jnp.int32)


def kernel(x):
    n, m = x.shape
    return pl.pallas_call(
        _onehot_block,
        grid=(n // _BLOCK_ROWS,),
        in_specs=[pl.BlockSpec((_BLOCK_ROWS, m), lambda i: (i, 0))],
        out_specs=pl.BlockSpec((_BLOCK_ROWS, m, _NUM_CLASSES),
                               lambda i: (i, 0, 0)),
        out_shape=jax.ShapeDtypeStruct((n, m, _NUM_CLASSES), jnp.int32),
    )(x)


# speedup vs baseline: 4.7606x; 4.7606x over previous
import jax
import jax.numpy as jnp
from jax.experimental import pallas as pl

_NUM_CLASSES = 805


def _onehot_block(x_ref, o_ref):
    j = pl.program_id(0)
    row = x_ref[pl.ds(j, 1), :]
    iota = jax.lax.broadcasted_iota(jnp.int32, o_ref.shape, 1)
    o_ref[...] = jnp.where(row[:, None, :] == iota, 1, 0)


def kernel(x):
    n, m = x.shape
    xt = x.T
    out_t = pl.pallas_call(
        _onehot_block,
        grid=(m,),
        in_specs=[pl.BlockSpec((m, n), lambda j: (0, 0))],
        out_specs=pl.BlockSpec((1, _NUM_CLASSES, n), lambda j: (j, 0, 0)),
        out_shape=jax.ShapeDtypeStruct((m, _NUM_CLASSES, n), jnp.int32),
    )(xt)
    return jnp.transpose(out_t, (2, 0, 1))
